# Initial kernel scaffold; baseline (speedup 1.0000x reference)
import jax
import jax.numpy as jnp
from jax.experimental import pallas as pl


def kernel(x_tokens, y_tokens, lane_indices, x_table, y_table, pos_table, lane_table):
    raise NotImplementedError("write your pallas kernel here")


# speedup vs baseline: 8.1314x; 8.1314x over previous
import functools

import jax
import jax.numpy as jnp
from jax import lax
from jax.experimental import pallas as pl
from jax.experimental.pallas import tpu as pltpu
from jax.experimental.pallas import tpu_sc as plsc

BATCH = 4096
SEQ = 200
DIM = 64
NUM_CORES = 2
NUM_SUBCORES = 16
NW = NUM_CORES * NUM_SUBCORES
BPW = BATCH // NW
SPLIT_A = 104
SPLIT_B = SEQ - SPLIT_A


def _body(x_tok, y_tok, lane_idx_hbm, x_tab, y_tab, p_tab, l_tab, out_hbm,
          xt_a, xt_b, yt_a, yt_b, lane_idx, lane_rows, pos_v, buf_x, buf_y,
          obuf, sem):
    cid = lax.axis_index("c")
    sid = lax.axis_index("s")
    wid = sid * NUM_CORES + cid
    base_b = wid * BPW

    pltpu.sync_copy(p_tab, pos_v)
    pltpu.sync_copy(lane_idx_hbm.at[pl.ds(base_b, BPW)], lane_idx)
    pltpu.async_copy(l_tab.at[lane_idx], lane_rows, sem).wait()

    def batch_body(j, carry):
        b = base_b + j
        off = pl.multiple_of(b * SEQ, 8)
        off2 = pl.multiple_of(b * SEQ + SPLIT_A, 8)
        pltpu.sync_copy(x_tok.at[pl.ds(off, SPLIT_A)], xt_a)
        pltpu.sync_copy(x_tok.at[pl.ds(off2, SPLIT_B)], xt_b)
        pltpu.sync_copy(y_tok.at[pl.ds(off, SPLIT_A)], yt_a)
        pltpu.sync_copy(y_tok.at[pl.ds(off2, SPLIT_B)], yt_b)

        c0 = pltpu.async_copy(x_tab.at[xt_a], buf_x.at[pl.ds(0, SPLIT_A)], sem)
        c1 = pltpu.async_copy(x_tab.at[xt_b], buf_x.at[pl.ds(SPLIT_A, SPLIT_B)], sem)
        c2 = pltpu.async_copy(y_tab.at[yt_a], buf_y.at[pl.ds(0, SPLIT_A)], sem)
        c3 = pltpu.async_copy(y_tab.at[yt_b], buf_y.at[pl.ds(SPLIT_A, SPLIT_B)], sem)
        c0.wait()
        c1.wait()
        c2.wait()
        c3.wait()

        lane_vecs = [lane_rows[j, pl.ds(q * 16, 16)] * 10.0 for q in range(4)]

        def row_body(r, rcarry):
            for q in range(4):
                sl = pl.ds(q * 16, 16)
                obuf[r, sl] = (buf_x[r, sl] + buf_y[r, sl] + pos_v[r, sl]
                               + lane_vecs[q])
            return rcarry

        lax.fori_loop(0, SEQ, row_body, 0)

        pltpu.sync_copy(obuf, out_hbm.at[b])
        return carry

    lax.fori_loop(0, BPW, batch_body, 0)


_sc_call = functools.partial(
    pl.kernel,
    mesh=plsc.VectorSubcoreMesh(core_axis_name="c", subcore_axis_name="s"),
    out_type=jax.ShapeDtypeStruct((BATCH, SEQ, DIM), jnp.float32),
    scratch_types=[
        pltpu.VMEM((SPLIT_A,), jnp.int32),
        pltpu.VMEM((SPLIT_B,), jnp.int32),
        pltpu.VMEM((SPLIT_A,), jnp.int32),
        pltpu.VMEM((SPLIT_B,), jnp.int32),
        pltpu.VMEM((BPW,), jnp.int32),
        pltpu.VMEM((BPW, 2 * DIM), jnp.float32),
        pltpu.VMEM((SEQ, DIM), jnp.float32),
        pltpu.VMEM((SEQ, 2 * DIM), jnp.float32),
        pltpu.VMEM((SEQ, 2 * DIM), jnp.float32),
        pltpu.VMEM((SEQ, DIM), jnp.float32),
        pltpu.SemaphoreType.DMA,
    ],
)(_body)


@jax.jit
def kernel(x_tokens, y_tokens, lane_indices, x_table, y_table, pos_table,
           lane_table):
    x_tokens = x_tokens.astype(jnp.int32).reshape(BATCH * SEQ)
    y_tokens = y_tokens.astype(jnp.int32).reshape(BATCH * SEQ)
    lane_indices = lane_indices.astype(jnp.int32)
    pad = lambda t: jnp.pad(t, ((0, 0), (0, DIM)))
    return _sc_call(x_tokens, y_tokens, lane_indices, pad(x_table),
                    pad(y_table), pos_table, pad(lane_table))


# speedup vs baseline: 8.4878x; 1.0438x over previous
import functools

import jax
import jax.numpy as jnp
from jax import lax
from jax.experimental import pallas as pl
from jax.experimental.pallas import tpu as pltpu
from jax.experimental.pallas import tpu_sc as plsc

BATCH = 4096
SEQ = 200
DIM = 64
NUM_CORES = 2
NUM_SUBCORES = 16
NW = NUM_CORES * NUM_SUBCORES
BPW = BATCH // NW
SPLIT_A = 104
SPLIT_B = SEQ - SPLIT_A


def _body(x_tok, y_tok, lane_idx_hbm, x_tab, y_tab, p_tab, l_tab, out_hbm,
          xt_a, xt_b, yt_a, yt_b, lane_idx, lane_rows, pos_v, buf_x, buf_y,
          obuf, sem):
    cid = lax.axis_index("c")
    sid = lax.axis_index("s")
    wid = sid * NUM_CORES + cid
    base_b = wid * BPW

    pltpu.sync_copy(p_tab, pos_v)
    pltpu.sync_copy(lane_idx_hbm.at[pl.ds(base_b, BPW)], lane_idx)
    pltpu.async_copy(l_tab.at[lane_idx], lane_rows, sem).wait()

    def batch_body(j, carry):
        b = base_b + j
        off = pl.multiple_of(b * SEQ, 8)
        off2 = pl.multiple_of(b * SEQ + SPLIT_A, 8)
        pltpu.sync_copy(x_tok.at[pl.ds(off, SPLIT_A)], xt_a)
        pltpu.sync_copy(x_tok.at[pl.ds(off2, SPLIT_B)], xt_b)
        pltpu.sync_copy(y_tok.at[pl.ds(off, SPLIT_A)], yt_a)
        pltpu.sync_copy(y_tok.at[pl.ds(off2, SPLIT_B)], yt_b)

        c0 = pltpu.async_copy(x_tab.at[xt_a], buf_x.at[pl.ds(0, SPLIT_A)], sem)
        c1 = pltpu.async_copy(x_tab.at[xt_b], buf_x.at[pl.ds(SPLIT_A, SPLIT_B)], sem)
        c2 = pltpu.async_copy(y_tab.at[yt_a], buf_y.at[pl.ds(0, SPLIT_A)], sem)
        c3 = pltpu.async_copy(y_tab.at[yt_b], buf_y.at[pl.ds(SPLIT_A, SPLIT_B)], sem)
        c0.wait()
        c1.wait()
        c2.wait()
        c3.wait()

        lane_vecs = [lane_rows[j, pl.ds(q * 16, 16)] * 10.0 for q in range(4)]

        def row_body(r, rcarry):
            for q in range(4):
                sl = pl.ds(q * 16, 16)
                obuf[r, sl] = (buf_x[r, sl] + buf_y[r, sl] + pos_v[r, sl]
                               + lane_vecs[q])
            return rcarry

        lax.fori_loop(0, SEQ, row_body, 0)

        pltpu.sync_copy(obuf, out_hbm.at[b])
        return carry

    lax.fori_loop(0, BPW, batch_body, 0)


_sc_call = functools.partial(
    pl.kernel,
    mesh=plsc.VectorSubcoreMesh(core_axis_name="c", subcore_axis_name="s"),
    out_type=jax.ShapeDtypeStruct((BATCH, SEQ, DIM), jnp.float32),
    scratch_types=[
        pltpu.VMEM((SPLIT_A,), jnp.int32),
        pltpu.VMEM((SPLIT_B,), jnp.int32),
        pltpu.VMEM((SPLIT_A,), jnp.int32),
        pltpu.VMEM((SPLIT_B,), jnp.int32),
        pltpu.VMEM((BPW,), jnp.int32),
        pltpu.VMEM((BPW, DIM), jnp.float32),
        pltpu.VMEM((SEQ, DIM), jnp.float32),
        pltpu.VMEM((SEQ, DIM), jnp.float32),
        pltpu.VMEM((SEQ, DIM), jnp.float32),
        pltpu.VMEM((SEQ, DIM), jnp.float32),
        pltpu.SemaphoreType.DMA,
    ],
    compiler_params=pltpu.CompilerParams(use_tc_tiling_on_sc=False),
)(_body)


@jax.jit
def kernel(x_tokens, y_tokens, lane_indices, x_table, y_table, pos_table,
           lane_table):
    x_tokens = x_tokens.astype(jnp.int32).reshape(BATCH * SEQ)
    y_tokens = y_tokens.astype(jnp.int32).reshape(BATCH * SEQ)
    lane_indices = lane_indices.astype(jnp.int32)
    return _sc_call(x_tokens, y_tokens, lane_indices, x_table, y_table,
                    pos_table, lane_table)


# speedup vs baseline: 8.7815x; 1.0346x over previous
import functools

import jax
import jax.numpy as jnp
from jax import lax
from jax.experimental import pallas as pl
from jax.experimental.pallas import tpu as pltpu
from jax.experimental.pallas import tpu_sc as plsc

BATCH = 4096
SEQ = 200
DIM = 64
NUM_CORES = 2
NUM_SUBCORES = 16
NW = NUM_CORES * NUM_SUBCORES
BPW = BATCH // NW
SPLIT_A = 104
SPLIT_B = SEQ - SPLIT_A


def _body(x_tok, y_tok, lane_idx_hbm, x_tab, y_tab, p_tab, l_tab, out_hbm,
          xt_ring, yt_ring, lane_idx, lane_rows, pos_v, buf_x, buf_y, obuf,
          sem_x0, sem_x1, sem_y0, sem_y1, sem_o0, sem_o1,
          sem_t0, sem_t1, sem_t2, sem_t3):
    cid = lax.axis_index("c")
    sid = lax.axis_index("s")
    wid = sid * NUM_CORES + cid
    base_b = wid * BPW

    sem_x = [sem_x0, sem_x1]
    sem_y = [sem_y0, sem_y1]
    sem_o = [sem_o0, sem_o1]
    sem_t = [sem_t0, sem_t1, sem_t2, sem_t3]

    pltpu.sync_copy(p_tab, pos_v)
    pltpu.sync_copy(lane_idx_hbm.at[pl.ds(base_b, BPW)], lane_idx)
    pltpu.async_copy(l_tab.at[lane_idx], lane_rows, sem_x0).wait()

    def tok_descs(j, t):
        off = pl.multiple_of((base_b + j) * SEQ, 8)
        return (
            pltpu.make_async_copy(x_tok.at[pl.ds(off, SEQ)],
                                  xt_ring.at[t], sem_t[t]),
            pltpu.make_async_copy(y_tok.at[pl.ds(off, SEQ)],
                                  yt_ring.at[t], sem_t[t]),
        )

    def gather_descs(t, p):
        return (
            pltpu.make_async_copy(x_tab.at[xt_ring.at[t, pl.ds(0, SPLIT_A)]],
                                  buf_x.at[p, pl.ds(0, SPLIT_A)], sem_x[p]),
            pltpu.make_async_copy(
                x_tab.at[xt_ring.at[t, pl.ds(SPLIT_A, SPLIT_B)]],
                buf_x.at[p, pl.ds(SPLIT_A, SPLIT_B)], sem_x[p]),
            pltpu.make_async_copy(y_tab.at[yt_ring.at[t, pl.ds(0, SPLIT_A)]],
                                  buf_y.at[p, pl.ds(0, SPLIT_A)], sem_y[p]),
            pltpu.make_async_copy(
                y_tab.at[yt_ring.at[t, pl.ds(SPLIT_A, SPLIT_B)]],
                buf_y.at[p, pl.ds(SPLIT_A, SPLIT_B)], sem_y[p]),
        )

    def out_desc(j, p):
        return pltpu.make_async_copy(obuf.at[p], out_hbm.at[base_b + j],
                                     sem_o[p])

    for t in range(4):
        for d in tok_descs(t, t):
            d.start()
    for t in range(2):
        sem = sem_t[t]
        for d in tok_descs(t, t):
            d.wait()
        for d in gather_descs(t, t):
            d.start()

    def batch_body(j, carry):
        p = lax.rem(j, 2)

        for ps in range(2):
            @pl.when(p == ps)
            def _():
                for d in gather_descs(0, ps):
                    d.wait()

                @pl.when(j >= 2)
                def _():
                    out_desc(j - 2, ps).wait()

                lane_vecs = [lane_rows[j, pl.ds(q * 16, 16)] * 10.0
                             for q in range(4)]

                def row_body(r, rcarry):
                    for q in range(4):
                        sl = pl.ds(q * 16, 16)
                        obuf[ps, r, sl] = (buf_x[ps, r, sl] + buf_y[ps, r, sl]
                                           + pos_v[r, sl] + lane_vecs[q])
                    return rcarry

                lax.fori_loop(0, SEQ, row_body, 0)

                out_desc(j, ps).start()

                @pl.when(j + 2 < BPW)
                def _():
                    t2 = lax.rem(j + 2, 4)
                    for ts in range(4):
                        @pl.when(t2 == ts)
                        def _():
                            for d in tok_descs(j + 2, ts):
                                d.wait()
                            for d in gather_descs(ts, ps):
                                d.start()

                @pl.when(j + 4 < BPW)
                def _():
                    t4 = lax.rem(j + 4, 4)
                    for ts in range(4):
                        @pl.when(t4 == ts)
                        def _():
                            for d in tok_descs(j + 4, ts):
                                d.start()
        return carry

    lax.fori_loop(0, BPW, batch_body, 0)

    out_desc(BPW - 2, 0).wait()
    out_desc(BPW - 1, 1).wait()


_sc_call = functools.partial(
    pl.kernel,
    mesh=plsc.VectorSubcoreMesh(core_axis_name="c", subcore_axis_name="s"),
    out_type=jax.ShapeDtypeStruct((BATCH, SEQ, DIM), jnp.float32),
    scratch_types=[
        pltpu.VMEM((4, SEQ), jnp.int32),
        pltpu.VMEM((4, SEQ), jnp.int32),
        pltpu.VMEM((BPW,), jnp.int32),
        pltpu.VMEM((BPW, DIM), jnp.float32),
        pltpu.VMEM((SEQ, DIM), jnp.float32),
        pltpu.VMEM((2, SEQ, DIM), jnp.float32),
        pltpu.VMEM((2, SEQ, DIM), jnp.float32),
        pltpu.VMEM((2, SEQ, DIM), jnp.float32),
        pltpu.SemaphoreType.DMA,
        pltpu.SemaphoreType.DMA,
        pltpu.SemaphoreType.DMA,
        pltpu.SemaphoreType.DMA,
        pltpu.SemaphoreType.DMA,
        pltpu.SemaphoreType.DMA,
        pltpu.SemaphoreType.DMA,
        pltpu.SemaphoreType.DMA,
        pltpu.SemaphoreType.DMA,
        pltpu.SemaphoreType.DMA,
    ],
    compiler_params=pltpu.CompilerParams(use_tc_tiling_on_sc=False),
)(_body)


@jax.jit
def kernel(x_tokens, y_tokens, lane_indices, x_table, y_table, pos_table,
           lane_table):
    x_tokens = x_tokens.astype(jnp.int32).reshape(BATCH * SEQ)
    y_tokens = y_tokens.astype(jnp.int32).reshape(BATCH * SEQ)
    lane_indices = lane_indices.astype(jnp.int32)
    return _sc_call(x_tokens, y_tokens, lane_indices, x_table, y_table,
                    pos_table, lane_table)
